# baseline (speedup 1.0000x reference)
import functools

import jax
import jax.numpy as jnp
from jax import lax
from jax.experimental import pallas as pl
from jax.experimental.pallas import tpu as pltpu
from jax.experimental.pallas import tpu_sc as plsc

NUM_CLASSES = 1000000
D = 64
B = 16384

NC = 2
NS = 16
NW = NC * NS
BPW = B // NW

LN_BLK = 2048


def _sc_gather(table, idx):
    mesh = plsc.VectorSubcoreMesh(core_axis_name="c", subcore_axis_name="s")

    @functools.partial(
        pl.kernel,
        mesh=mesh,
        out_type=jax.ShapeDtypeStruct((B, D), jnp.float32),
        scratch_types=[
            pltpu.VMEM((BPW,), jnp.int32),
            pltpu.VMEM((BPW, D), jnp.float32),
            pltpu.SemaphoreType.DMA,
        ],
        compiler_params=pltpu.CompilerParams(use_tc_tiling_on_sc=False),
    )
    def k(table_hbm, idx_hbm, out_hbm, idx_v, rows_v, sem):
        wid = lax.axis_index("s") * NC + lax.axis_index("c")
        base = wid * BPW
        pltpu.sync_copy(idx_hbm.at[pl.ds(base, BPW)], idx_v)
        pltpu.async_copy(table_hbm.at[idx_v], rows_v, sem).wait()
        pltpu.sync_copy(rows_v, out_hbm.at[pl.ds(base, BPW)])

    return k(table, idx)


def _ln_body(x_ref, w_ref, b_ref, o_ref):
    x = x_ref[...]
    mean = jnp.mean(x, axis=-1, keepdims=True)
    var = jnp.mean((x - mean) ** 2, axis=-1, keepdims=True)
    o_ref[...] = (x - mean) * lax.rsqrt(var + 1e-5) * w_ref[...] + b_ref[...]


def _tc_layernorm(x, w, b):
    return pl.pallas_call(
        _ln_body,
        out_shape=jax.ShapeDtypeStruct((B, D), jnp.float32),
        grid=(B // LN_BLK,),
        in_specs=[
            pl.BlockSpec((LN_BLK, D), lambda i: (i, 0)),
            pl.BlockSpec((1, D), lambda i: (0, 0)),
            pl.BlockSpec((1, D), lambda i: (0, 0)),
        ],
        out_specs=pl.BlockSpec((LN_BLK, D), lambda i: (i, 0)),
    )(x, w.reshape(1, D), b.reshape(1, D))


def kernel(class_labels, table, ln_w, ln_b):
    rows = _sc_gather(table, class_labels.astype(jnp.int32))
    y = _tc_layernorm(rows, ln_w, ln_b)
    return y[:, None, :]
